# baseline (speedup 1.0000x reference)
import functools

import numpy as np
import jax
import jax.numpy as jnp
from jax import lax
from jax.experimental import pallas as pl
from jax.experimental.pallas import tpu as pltpu
from jax.experimental.pallas import tpu_sc as plsc

D = 1024
FF = 2816
E = 64
TOPK = 8
R = 16
SCALING = 2.0

FT = 256
NFT = FF // FT
HFT = FT // 2
FH = FF // 2
DH = D // 2
T = 128
NT = 192
P = NT * T
S_TOK = 2048
NPAIR = S_TOK * TOPK

NC, NS = 2, 16
NW = NC * NS

_PERM_FF = np.concatenate([
    (np.arange(NFT)[:, None] * FT + np.arange(HFT)[None, :]).reshape(-1),
    (np.arange(NFT)[:, None] * FT + HFT + np.arange(HFT)[None, :]).reshape(-1),
])


def _pack(lo, hi):
    lo_u = lax.bitcast_convert_type(
        lo.astype(jnp.bfloat16).astype(jnp.float32), jnp.uint32)
    hi_u = lax.bitcast_convert_type(
        hi.astype(jnp.bfloat16).astype(jnp.float32), jnp.uint32)
    packed = (hi_u & jnp.uint32(0xFFFF0000)) | (lo_u >> 16)
    return lax.bitcast_convert_type(packed, jnp.int32)


def _unpack(packed_i32):
    u = lax.bitcast_convert_type(packed_i32, jnp.uint32)
    lo = lax.bitcast_convert_type(u << 16, jnp.float32)
    hi = lax.bitcast_convert_type(u & jnp.uint32(0xFFFF0000), jnp.float32)
    return jnp.concatenate([lo, hi], axis=1)


def _k1_body(x_ref, wg_ref, wu_ref, wd_ref, wr_ref,
             gp_ref, up_ref, xp_ref, out_ref, logits_ref):
    f = pl.program_id(0)
    x = x_ref[...]
    xb = x.astype(jnp.bfloat16)
    g = jax.lax.dot_general(xb, wg_ref[...], (((1,), (1,)), ((), ())),
                            preferred_element_type=jnp.float32)
    u = jax.lax.dot_general(xb, wu_ref[...], (((1,), (1,)), ((), ())),
                            preferred_element_type=jnp.float32)
    gp_ref[...] = _pack(g[:, :HFT], g[:, HFT:])
    up_ref[...] = _pack(u[:, :HFT], u[:, HFT:])
    h = ((g / (1.0 + jnp.exp(-g))) * u).astype(jnp.bfloat16)
    part = jax.lax.dot_general(h, wd_ref[...], (((1,), (1,)), ((), ())),
                               preferred_element_type=jnp.float32)

    @pl.when(f == 0)
    def _():
        out_ref[...] = part
        logits_ref[...] = jax.lax.dot_general(
            xb, wr_ref[...], (((1,), (1,)), ((), ())),
            preferred_element_type=jnp.float32)
        xp_ref[...] = _pack(x[:, :DH], x[:, DH:])

    @pl.when(f != 0)
    def _():
        out_ref[...] += part


def _base_mlp(xf, Wgb, Wub, Wdb, Wrb):
    S = xf.shape[0]
    return pl.pallas_call(
        _k1_body,
        grid=(NFT,),
        in_specs=[
            pl.BlockSpec((S, D), lambda f: (0, 0)),
            pl.BlockSpec((FT, D), lambda f: (f, 0)),
            pl.BlockSpec((FT, D), lambda f: (f, 0)),
            pl.BlockSpec((D, FT), lambda f: (0, f)),
            pl.BlockSpec((E, D), lambda f: (0, 0)),
        ],
        out_specs=[
            pl.BlockSpec((S, HFT), lambda f: (0, f)),
            pl.BlockSpec((S, HFT), lambda f: (0, f)),
            pl.BlockSpec((S, DH), lambda f: (0, 0)),
            pl.BlockSpec((S, D), lambda f: (0, 0)),
            pl.BlockSpec((S, E), lambda f: (0, 0)),
        ],
        out_shape=[
            jax.ShapeDtypeStruct((S, FH), jnp.int32),
            jax.ShapeDtypeStruct((S, FH), jnp.int32),
            jax.ShapeDtypeStruct((S, DH), jnp.int32),
            jax.ShapeDtypeStruct((S, D), jnp.float32),
            jax.ShapeDtypeStruct((S, E), jnp.float32),
        ],
    )(xf, Wgb, Wub, Wdb, Wrb)


def _dispatch(sel, rw):
    S = sel.shape[0]
    onehot = (sel[:, :, None] == jnp.arange(E, dtype=sel.dtype)[None, None, :])
    onehot = onehot.any(axis=1).astype(jnp.int32)
    cumincl = jnp.cumsum(onehot, axis=0)
    counts = cumincl[-1]
    cumexcl = cumincl - onehot
    padded = ((counts + T - 1) // T) * T
    pend = jnp.cumsum(padded)
    poff = (pend - padded).astype(jnp.int32)
    rank = jnp.take_along_axis(cumexcl, sel, axis=1)
    ppos = poff[sel] + rank.astype(jnp.int32)
    tile_expert = jnp.searchsorted(
        pend, jnp.arange(NT, dtype=jnp.int32) * T, side='right')
    tile_expert = jnp.minimum(tile_expert, E - 1).astype(jnp.int32)
    ppos_flat = ppos.reshape(-1)
    slot_token = jnp.zeros((P,), jnp.int32).at[ppos_flat].set(
        jnp.arange(NPAIR, dtype=jnp.int32) // TOPK)
    slot_w = jnp.zeros((P,), jnp.float32).at[ppos_flat].set(rw.reshape(-1))
    return tile_expert, slot_token, slot_w, ppos_flat


_K2_CH = 32
_K2_PER_W = P // NW


def _k2_pipelined_table(tbl_hbm, out_hbm, idx_v, base, bufs, sems):
    n = _K2_PER_W // _K2_CH
    copies = [None, None]
    copies[0] = pltpu.make_async_copy(
        tbl_hbm.at[idx_v.at[pl.ds(0, _K2_CH)]], bufs[0], sems[0])
    copies[0].start()
    for i in range(n):
        par = i % 2
        nxt = 1 - par
        copies[par].wait()
        if i + 1 < n:
            copies[nxt] = pltpu.make_async_copy(
                tbl_hbm.at[idx_v.at[pl.ds((i + 1) * _K2_CH, _K2_CH)]],
                bufs[nxt], sems[nxt])
            copies[nxt].start()
        pltpu.sync_copy(bufs[par],
                        out_hbm.at[pl.ds(base + i * _K2_CH, _K2_CH)])


def _k2_body(gp_hbm, up_hbm, xp_hbm, tok_hbm,
             gs_hbm, us_hbm, xs_hbm,
             idx_v, ga_v, gb_v, xa_v, xb_v, s0, s1):
    wid = lax.axis_index("s") * NC + lax.axis_index("c")
    base = wid * _K2_PER_W
    pltpu.sync_copy(tok_hbm.at[pl.ds(base, _K2_PER_W)], idx_v)
    _k2_pipelined_table(gp_hbm, gs_hbm, idx_v, base, (ga_v, gb_v), (s0, s1))
    _k2_pipelined_table(up_hbm, us_hbm, idx_v, base, (ga_v, gb_v), (s0, s1))
    _k2_pipelined_table(xp_hbm, xs_hbm, idx_v, base, (xa_v, xb_v), (s0, s1))


def _stage_gather(gate_p, up_p, x_p, slot_token):
    mesh = plsc.VectorSubcoreMesh(core_axis_name="c", subcore_axis_name="s")
    return pl.kernel(
        _k2_body,
        out_type=(
            jax.ShapeDtypeStruct((P, FH), jnp.int32),
            jax.ShapeDtypeStruct((P, FH), jnp.int32),
            jax.ShapeDtypeStruct((P, DH), jnp.int32),
        ),
        mesh=mesh,
        scratch_types=[
            pltpu.VMEM((_K2_PER_W,), jnp.int32),
            pltpu.VMEM((_K2_CH, FH), jnp.int32),
            pltpu.VMEM((_K2_CH, FH), jnp.int32),
            pltpu.VMEM((_K2_CH, DH), jnp.int32),
            pltpu.VMEM((_K2_CH, DH), jnp.int32),
            pltpu.SemaphoreType.DMA,
            pltpu.SemaphoreType.DMA,
        ],
    )(gate_p, up_p, x_p, slot_token)


def _k3_body(te_ref, gs_ref, us_ref, xs_ref, w_ref,
             ag_ref, bgt_ref, au_ref, but_ref, ad_ref, bdt_ref,
             delta_ref):
    xs = _unpack(xs_ref[...]).astype(jnp.bfloat16)
    xag = jax.lax.dot_general(xs, ag_ref[...], (((1,), (1,)), ((), ())),
                              preferred_element_type=jnp.float32)
    gd = jax.lax.dot_general(xag.astype(jnp.bfloat16), bgt_ref[...],
                             (((1,), (0,)), ((), ())),
                             preferred_element_type=jnp.float32)
    xau = jax.lax.dot_general(xs, au_ref[...], (((1,), (1,)), ((), ())),
                              preferred_element_type=jnp.float32)
    ud = jax.lax.dot_general(xau.astype(jnp.bfloat16), but_ref[...],
                             (((1,), (0,)), ((), ())),
                             preferred_element_type=jnp.float32)
    gate = _unpack(gs_ref[...]) + SCALING * gd
    up = _unpack(us_ref[...]) + SCALING * ud
    hidden = ((gate / (1.0 + jnp.exp(-gate))) * up).astype(jnp.bfloat16)
    had = jax.lax.dot_general(hidden, ad_ref[...], (((1,), (1,)), ((), ())),
                              preferred_element_type=jnp.float32)
    had = (had * (SCALING * w_ref[...])).astype(jnp.bfloat16)
    delta_ref[...] = jax.lax.dot_general(
        had, bdt_ref[...], (((1,), (0,)), ((), ())),
        preferred_element_type=jnp.float32)


def _expert_deltas(tile_expert, gs_s, us_s, xs_s, w_col,
                   Agb, BgTb, Aub, BuTb, Adb, BdTb):
    grid_spec = pltpu.PrefetchScalarGridSpec(
        num_scalar_prefetch=1,
        grid=(NT,),
        in_specs=[
            pl.BlockSpec((T, FH), lambda j, te: (j, 0)),
            pl.BlockSpec((T, FH), lambda j, te: (j, 0)),
            pl.BlockSpec((T, DH), lambda j, te: (j, 0)),
            pl.BlockSpec((T, 1), lambda j, te: (j, 0)),
            pl.BlockSpec((None, R, D), lambda j, te: (te[j], 0, 0)),
            pl.BlockSpec((None, R, FF), lambda j, te: (te[j], 0, 0)),
            pl.BlockSpec((None, R, D), lambda j, te: (te[j], 0, 0)),
            pl.BlockSpec((None, R, FF), lambda j, te: (te[j], 0, 0)),
            pl.BlockSpec((None, R, FF), lambda j, te: (te[j], 0, 0)),
            pl.BlockSpec((None, R, D), lambda j, te: (te[j], 0, 0)),
        ],
        out_specs=pl.BlockSpec((T, D), lambda j, te: (j, 0)),
    )
    return pl.pallas_call(
        _k3_body,
        grid_spec=grid_spec,
        out_shape=jax.ShapeDtypeStruct((P, D), jnp.float32),
    )(tile_expert, gs_s, us_s, xs_s, w_col,
      Agb, BgTb, Aub, BuTb, Adb, BdTb)


_K4_CH = 64
_K4_PER_W = NPAIR // NW


def _k4_body(delta_hbm, pos_hbm, dt_hbm, idx_v, rows_v, sem):
    wid = lax.axis_index("s") * NC + lax.axis_index("c")
    base = wid * _K4_PER_W

    def chunk(i, _):
        off = base + i * _K4_CH
        pltpu.sync_copy(pos_hbm.at[pl.ds(off, _K4_CH)], idx_v)
        pltpu.async_copy(delta_hbm.at[idx_v], rows_v, sem).wait()
        pltpu.sync_copy(rows_v, dt_hbm.at[pl.ds(off, _K4_CH)])
        return ()

    lax.fori_loop(0, _K4_PER_W // _K4_CH, chunk, ())


def _delta_to_token_order(delta, ppos_flat):
    mesh = plsc.VectorSubcoreMesh(core_axis_name="c", subcore_axis_name="s")
    return pl.kernel(
        _k4_body,
        out_type=jax.ShapeDtypeStruct((NPAIR, D), jnp.float32),
        mesh=mesh,
        scratch_types=[
            pltpu.VMEM((_K4_CH,), jnp.int32),
            pltpu.VMEM((_K4_CH, D), jnp.float32),
            pltpu.SemaphoreType.DMA,
        ],
    )(delta, ppos_flat)


_K5_T = 128


def _k5_body(dt_ref, base_ref, out_ref):
    d = dt_ref[...].reshape(_K5_T, TOPK, D)
    out_ref[...] = base_ref[...] + d.sum(axis=1)


def _combine(delta_tok, base_out):
    S = base_out.shape[0]
    return pl.pallas_call(
        _k5_body,
        grid=(S // _K5_T,),
        in_specs=[
            pl.BlockSpec((_K5_T * TOPK, D), lambda i: (i, 0)),
            pl.BlockSpec((_K5_T, D), lambda i: (i, 0)),
        ],
        out_specs=pl.BlockSpec((_K5_T, D), lambda i: (i, 0)),
        out_shape=jax.ShapeDtypeStruct((S, D), jnp.float32),
    )(delta_tok, base_out)


def kernel(x, Wg, Wu, Wd, Wr, Ag, Bg, Au, Bu, Ad, Bd):
    b, s, d = x.shape
    xf = x.reshape(-1, d)

    gate_p, up_p, x_p, base_out, logits = _base_mlp(
        xf, Wg.astype(jnp.bfloat16), Wu.astype(jnp.bfloat16),
        Wd.astype(jnp.bfloat16), Wr.astype(jnp.bfloat16))

    probs = jax.nn.softmax(logits, axis=-1)
    rw, sel = jax.lax.top_k(probs, TOPK)
    tile_expert, slot_token, slot_w, ppos_flat = _dispatch(sel, rw)

    gs_s, us_s, xs_s = _stage_gather(gate_p, up_p, x_p, slot_token)

    perm = jnp.asarray(_PERM_FF)
    Agb = Ag.astype(jnp.bfloat16)
    Aub = Au.astype(jnp.bfloat16)
    Adb = Ad[:, :, perm].astype(jnp.bfloat16)
    BgTb = jnp.swapaxes(Bg, 1, 2)[:, :, perm].astype(jnp.bfloat16)
    BuTb = jnp.swapaxes(Bu, 1, 2)[:, :, perm].astype(jnp.bfloat16)
    BdTb = jnp.swapaxes(Bd, 1, 2).astype(jnp.bfloat16)

    delta = _expert_deltas(tile_expert, gs_s, us_s, xs_s,
                           slot_w.reshape(P, 1), Agb, BgTb, Aub, BuTb,
                           Adb, BdTb)

    delta_tok = _delta_to_token_order(delta, ppos_flat)
    return _combine(delta_tok, base_out).reshape(b, s, d)


# speedup vs baseline: 1.2373x; 1.2373x over previous
import functools

import numpy as np
import jax
import jax.numpy as jnp
from jax import lax
from jax.experimental import pallas as pl
from jax.experimental.pallas import tpu as pltpu
from jax.experimental.pallas import tpu_sc as plsc

D = 1024
FF = 2816
E = 64
TOPK = 8
R = 16
SCALING = 2.0

FT = 256
NFT = FF // FT
HFT = FT // 2
FH = FF // 2
DH = D // 2
T = 128
NT = 192
P = NT * T
S_TOK = 2048
NPAIR = S_TOK * TOPK

NC, NS = 2, 16
NW = NC * NS

_PERM_FF = np.concatenate([
    (np.arange(NFT)[:, None] * FT + np.arange(HFT)[None, :]).reshape(-1),
    (np.arange(NFT)[:, None] * FT + HFT + np.arange(HFT)[None, :]).reshape(-1),
])


def _pack(lo, hi):
    lo_u = lax.bitcast_convert_type(
        lo.astype(jnp.bfloat16).astype(jnp.float32), jnp.uint32)
    hi_u = lax.bitcast_convert_type(
        hi.astype(jnp.bfloat16).astype(jnp.float32), jnp.uint32)
    packed = (hi_u & jnp.uint32(0xFFFF0000)) | (lo_u >> 16)
    return lax.bitcast_convert_type(packed, jnp.int32)


def _unpack(packed_i32):
    u = lax.bitcast_convert_type(packed_i32, jnp.uint32)
    lo = lax.bitcast_convert_type(u << 16, jnp.float32)
    hi = lax.bitcast_convert_type(u & jnp.uint32(0xFFFF0000), jnp.float32)
    return jnp.concatenate([lo, hi], axis=1)


def _k1_body(x_ref, wg_ref, wu_ref, wd_ref, wr_ref,
             gp_ref, up_ref, xp_ref, out_ref, logits_ref):
    f = pl.program_id(0)
    x = x_ref[...]
    xb = x.astype(jnp.bfloat16)
    g = jax.lax.dot_general(xb, wg_ref[...], (((1,), (1,)), ((), ())),
                            preferred_element_type=jnp.float32)
    u = jax.lax.dot_general(xb, wu_ref[...], (((1,), (1,)), ((), ())),
                            preferred_element_type=jnp.float32)
    gp_ref[...] = _pack(g[:, :HFT], g[:, HFT:])
    up_ref[...] = _pack(u[:, :HFT], u[:, HFT:])
    h = ((g / (1.0 + jnp.exp(-g))) * u).astype(jnp.bfloat16)
    part = jax.lax.dot_general(h, wd_ref[...], (((1,), (1,)), ((), ())),
                               preferred_element_type=jnp.float32)

    @pl.when(f == 0)
    def _():
        out_ref[...] = part
        logits_ref[...] = jax.lax.dot_general(
            xb, wr_ref[...], (((1,), (1,)), ((), ())),
            preferred_element_type=jnp.float32)
        xp_ref[...] = _pack(x[:, :DH], x[:, DH:])

    @pl.when(f != 0)
    def _():
        out_ref[...] += part


def _base_mlp(xf, Wgb, Wub, Wdb, Wrb):
    S = xf.shape[0]
    return pl.pallas_call(
        _k1_body,
        grid=(NFT,),
        in_specs=[
            pl.BlockSpec((S, D), lambda f: (0, 0)),
            pl.BlockSpec((FT, D), lambda f: (f, 0)),
            pl.BlockSpec((FT, D), lambda f: (f, 0)),
            pl.BlockSpec((D, FT), lambda f: (0, f)),
            pl.BlockSpec((E, D), lambda f: (0, 0)),
        ],
        out_specs=[
            pl.BlockSpec((S, HFT), lambda f: (0, f)),
            pl.BlockSpec((S, HFT), lambda f: (0, f)),
            pl.BlockSpec((S, DH), lambda f: (0, 0)),
            pl.BlockSpec((S, D), lambda f: (0, 0)),
            pl.BlockSpec((S, E), lambda f: (0, 0)),
        ],
        out_shape=[
            jax.ShapeDtypeStruct((S, FH), jnp.int32),
            jax.ShapeDtypeStruct((S, FH), jnp.int32),
            jax.ShapeDtypeStruct((S, DH), jnp.int32),
            jax.ShapeDtypeStruct((S, D), jnp.float32),
            jax.ShapeDtypeStruct((S, E), jnp.float32),
        ],
    )(xf, Wgb, Wub, Wdb, Wrb)


def _dispatch(sel, rw):
    S = sel.shape[0]
    onehot = (sel[:, :, None] == jnp.arange(E, dtype=sel.dtype)[None, None, :])
    onehot = onehot.any(axis=1).astype(jnp.int32)
    cumincl = jnp.cumsum(onehot, axis=0)
    counts = cumincl[-1]
    cumexcl = cumincl - onehot
    padded = ((counts + T - 1) // T) * T
    pend = jnp.cumsum(padded)
    poff = (pend - padded).astype(jnp.int32)
    rank = jnp.take_along_axis(cumexcl, sel, axis=1)
    ppos = poff[sel] + rank.astype(jnp.int32)
    tile_expert = jnp.searchsorted(
        pend, jnp.arange(NT, dtype=jnp.int32) * T, side='right')
    tile_expert = jnp.minimum(tile_expert, E - 1).astype(jnp.int32)
    ppos_flat = ppos.reshape(-1)
    slot_token = jnp.zeros((P,), jnp.int32).at[ppos_flat].set(
        jnp.arange(NPAIR, dtype=jnp.int32) // TOPK)
    slot_w = jnp.zeros((P,), jnp.float32).at[ppos_flat].set(rw.reshape(-1))
    return tile_expert, slot_token, slot_w, ppos_flat


_K2_CH = 16
_K2_PER_W = P // NW


def _k2_body(gp_hbm, up_hbm, xp_hbm, tok_hbm,
             gs_hbm, us_hbm, xs_hbm,
             idx0, idx1, g0, g1, u0, u1, x0, x1,
             sg0, sg1, su0, su1, sx0, sx1):
    wid = lax.axis_index("s") * NC + lax.axis_index("c")
    base = wid * _K2_PER_W
    n = _K2_PER_W // _K2_CH
    idx = (idx0, idx1)
    gb = (g0, g1)
    ub = (u0, u1)
    xb = (x0, x1)
    sg = (sg0, sg1)
    su = (su0, su1)
    sx = (sx0, sx1)

    def start(i):
        par = i % 2
        off = base + i * _K2_CH
        pltpu.sync_copy(tok_hbm.at[pl.ds(off, _K2_CH)], idx[par])
        cg = pltpu.make_async_copy(gp_hbm.at[idx[par]], gb[par], sg[par])
        cu = pltpu.make_async_copy(up_hbm.at[idx[par]], ub[par], su[par])
        cx = pltpu.make_async_copy(xp_hbm.at[idx[par]], xb[par], sx[par])
        cg.start(); cu.start(); cx.start()
        return cg, cu, cx

    pend = start(0)
    for i in range(n):
        par = i % 2
        off = base + i * _K2_CH
        cg, cu, cx = pend
        cg.wait(); cu.wait(); cx.wait()
        if i + 1 < n:
            pend = start(i + 1)
        pltpu.sync_copy(gb[par], gs_hbm.at[pl.ds(off, _K2_CH)])
        pltpu.sync_copy(ub[par], us_hbm.at[pl.ds(off, _K2_CH)])
        pltpu.sync_copy(xb[par], xs_hbm.at[pl.ds(off, _K2_CH)])


def _stage_gather(gate_p, up_p, x_p, slot_token):
    mesh = plsc.VectorSubcoreMesh(core_axis_name="c", subcore_axis_name="s")
    return pl.kernel(
        _k2_body,
        out_type=(
            jax.ShapeDtypeStruct((P, FH), jnp.int32),
            jax.ShapeDtypeStruct((P, FH), jnp.int32),
            jax.ShapeDtypeStruct((P, DH), jnp.int32),
        ),
        mesh=mesh,
        scratch_types=[
            pltpu.VMEM((_K2_CH,), jnp.int32),
            pltpu.VMEM((_K2_CH,), jnp.int32),
            pltpu.VMEM((_K2_CH, FH), jnp.int32),
            pltpu.VMEM((_K2_CH, FH), jnp.int32),
            pltpu.VMEM((_K2_CH, FH), jnp.int32),
            pltpu.VMEM((_K2_CH, FH), jnp.int32),
            pltpu.VMEM((_K2_CH, DH), jnp.int32),
            pltpu.VMEM((_K2_CH, DH), jnp.int32),
            pltpu.SemaphoreType.DMA,
            pltpu.SemaphoreType.DMA,
            pltpu.SemaphoreType.DMA,
            pltpu.SemaphoreType.DMA,
            pltpu.SemaphoreType.DMA,
            pltpu.SemaphoreType.DMA,
        ],
    )(gate_p, up_p, x_p, slot_token)


def _k3_body(te_ref, gs_ref, us_ref, xs_ref, w_ref,
             ag_ref, bgt_ref, au_ref, but_ref, ad_ref, bdt_ref,
             delta_ref):
    xs = _unpack(xs_ref[...]).astype(jnp.bfloat16)
    xag = jax.lax.dot_general(xs, ag_ref[...], (((1,), (1,)), ((), ())),
                              preferred_element_type=jnp.float32)
    gd = jax.lax.dot_general(xag.astype(jnp.bfloat16), bgt_ref[...],
                             (((1,), (0,)), ((), ())),
                             preferred_element_type=jnp.float32)
    xau = jax.lax.dot_general(xs, au_ref[...], (((1,), (1,)), ((), ())),
                              preferred_element_type=jnp.float32)
    ud = jax.lax.dot_general(xau.astype(jnp.bfloat16), but_ref[...],
                             (((1,), (0,)), ((), ())),
                             preferred_element_type=jnp.float32)
    gate = _unpack(gs_ref[...]) + SCALING * gd
    up = _unpack(us_ref[...]) + SCALING * ud
    hidden = ((gate / (1.0 + jnp.exp(-gate))) * up).astype(jnp.bfloat16)
    had = jax.lax.dot_general(hidden, ad_ref[...], (((1,), (1,)), ((), ())),
                              preferred_element_type=jnp.float32)
    had = (had * (SCALING * w_ref[...])).astype(jnp.bfloat16)
    delta_ref[...] = jax.lax.dot_general(
        had, bdt_ref[...], (((1,), (0,)), ((), ())),
        preferred_element_type=jnp.float32)


def _expert_deltas(tile_expert, gs_s, us_s, xs_s, w_col,
                   Agb, BgTb, Aub, BuTb, Adb, BdTb):
    grid_spec = pltpu.PrefetchScalarGridSpec(
        num_scalar_prefetch=1,
        grid=(NT,),
        in_specs=[
            pl.BlockSpec((T, FH), lambda j, te: (j, 0)),
            pl.BlockSpec((T, FH), lambda j, te: (j, 0)),
            pl.BlockSpec((T, DH), lambda j, te: (j, 0)),
            pl.BlockSpec((T, 1), lambda j, te: (j, 0)),
            pl.BlockSpec((None, R, D), lambda j, te: (te[j], 0, 0)),
            pl.BlockSpec((None, R, FF), lambda j, te: (te[j], 0, 0)),
            pl.BlockSpec((None, R, D), lambda j, te: (te[j], 0, 0)),
            pl.BlockSpec((None, R, FF), lambda j, te: (te[j], 0, 0)),
            pl.BlockSpec((None, R, FF), lambda j, te: (te[j], 0, 0)),
            pl.BlockSpec((None, R, D), lambda j, te: (te[j], 0, 0)),
        ],
        out_specs=pl.BlockSpec((T, D), lambda j, te: (j, 0)),
    )
    return pl.pallas_call(
        _k3_body,
        grid_spec=grid_spec,
        out_shape=jax.ShapeDtypeStruct((P, D), jnp.float32),
    )(tile_expert, gs_s, us_s, xs_s, w_col,
      Agb, BgTb, Aub, BuTb, Adb, BdTb)


_K4_CH = 64
_K4_PER_W = NPAIR // NW


def _k4_body(delta_hbm, pos_hbm, dt_hbm, idx_v, rows_v, sem):
    wid = lax.axis_index("s") * NC + lax.axis_index("c")
    base = wid * _K4_PER_W

    def chunk(i, _):
        off = base + i * _K4_CH
        pltpu.sync_copy(pos_hbm.at[pl.ds(off, _K4_CH)], idx_v)
        pltpu.async_copy(delta_hbm.at[idx_v], rows_v, sem).wait()
        pltpu.sync_copy(rows_v, dt_hbm.at[pl.ds(off, _K4_CH)])
        return ()

    lax.fori_loop(0, _K4_PER_W // _K4_CH, chunk, ())


def _delta_to_token_order(delta, ppos_flat):
    mesh = plsc.VectorSubcoreMesh(core_axis_name="c", subcore_axis_name="s")
    return pl.kernel(
        _k4_body,
        out_type=jax.ShapeDtypeStruct((NPAIR, D), jnp.float32),
        mesh=mesh,
        scratch_types=[
            pltpu.VMEM((_K4_CH,), jnp.int32),
            pltpu.VMEM((_K4_CH, D), jnp.float32),
            pltpu.SemaphoreType.DMA,
        ],
    )(delta, ppos_flat)


_K5_T = 128


def _k5_body(dt_ref, base_ref, out_ref):
    d = dt_ref[...].reshape(_K5_T, TOPK, D)
    out_ref[...] = base_ref[...] + d.sum(axis=1)


def _combine(delta_tok, base_out):
    S = base_out.shape[0]
    return pl.pallas_call(
        _k5_body,
        grid=(S // _K5_T,),
        in_specs=[
            pl.BlockSpec((_K5_T * TOPK, D), lambda i: (i, 0)),
            pl.BlockSpec((_K5_T, D), lambda i: (i, 0)),
        ],
        out_specs=pl.BlockSpec((_K5_T, D), lambda i: (i, 0)),
        out_shape=jax.ShapeDtypeStruct((S, D), jnp.float32),
    )(delta_tok, base_out)


def kernel(x, Wg, Wu, Wd, Wr, Ag, Bg, Au, Bu, Ad, Bd):
    b, s, d = x.shape
    xf = x.reshape(-1, d)

    gate_p, up_p, x_p, base_out, logits = _base_mlp(
        xf, Wg.astype(jnp.bfloat16), Wu.astype(jnp.bfloat16),
        Wd.astype(jnp.bfloat16), Wr.astype(jnp.bfloat16))

    probs = jax.nn.softmax(logits, axis=-1)
    rw, sel = jax.lax.top_k(probs, TOPK)
    tile_expert, slot_token, slot_w, ppos_flat = _dispatch(sel, rw)

    gs_s, us_s, xs_s = _stage_gather(gate_p, up_p, x_p, slot_token)

    perm = jnp.asarray(_PERM_FF)
    Agb = Ag.astype(jnp.bfloat16)
    Aub = Au.astype(jnp.bfloat16)
    Adb = Ad[:, :, perm].astype(jnp.bfloat16)
    BgTb = jnp.swapaxes(Bg, 1, 2)[:, :, perm].astype(jnp.bfloat16)
    BuTb = jnp.swapaxes(Bu, 1, 2)[:, :, perm].astype(jnp.bfloat16)
    BdTb = jnp.swapaxes(Bd, 1, 2).astype(jnp.bfloat16)

    delta = _expert_deltas(tile_expert, gs_s, us_s, xs_s,
                           slot_w.reshape(P, 1), Agb, BgTb, Aub, BuTb,
                           Adb, BdTb)

    delta_tok = _delta_to_token_order(delta, ppos_flat)
    return _combine(delta_tok, base_out).reshape(b, s, d)


# speedup vs baseline: 1.9801x; 1.6003x over previous
import functools

import numpy as np
import jax
import jax.numpy as jnp
from jax import lax
from jax.experimental import pallas as pl
from jax.experimental.pallas import tpu as pltpu
from jax.experimental.pallas import tpu_sc as plsc

D = 1024
FF = 2816
E = 64
TOPK = 8
R = 16
SCALING = 2.0

FT = 256
NFT = FF // FT
HFT = FT // 2
FH = FF // 2
DH = D // 2
T = 128
NT = 192
P = NT * T
S_TOK = 2048
NPAIR = S_TOK * TOPK

NC, NS = 2, 16
NW = NC * NS

_PERM_FF = np.concatenate([
    (np.arange(NFT)[:, None] * FT + np.arange(HFT)[None, :]).reshape(-1),
    (np.arange(NFT)[:, None] * FT + HFT + np.arange(HFT)[None, :]).reshape(-1),
])


def _pack(lo, hi):
    lo_u = lax.bitcast_convert_type(
        lo.astype(jnp.bfloat16).astype(jnp.float32), jnp.uint32)
    hi_u = lax.bitcast_convert_type(
        hi.astype(jnp.bfloat16).astype(jnp.float32), jnp.uint32)
    packed = (hi_u & jnp.uint32(0xFFFF0000)) | (lo_u >> 16)
    return lax.bitcast_convert_type(packed, jnp.int32)


def _unpack(packed_i32):
    u = lax.bitcast_convert_type(packed_i32, jnp.uint32)
    lo = lax.bitcast_convert_type(u << 16, jnp.float32)
    hi = lax.bitcast_convert_type(u & jnp.uint32(0xFFFF0000), jnp.float32)
    return jnp.concatenate([lo, hi], axis=1)


def _k1_body(x_ref, wg_ref, wu_ref, wd_ref, wr_ref,
             gp_ref, up_ref, xp_ref, out_ref, logits_ref):
    f = pl.program_id(0)
    x = x_ref[...]
    xb = x.astype(jnp.bfloat16)
    g = jax.lax.dot_general(xb, wg_ref[...], (((1,), (1,)), ((), ())),
                            preferred_element_type=jnp.float32)
    u = jax.lax.dot_general(xb, wu_ref[...], (((1,), (1,)), ((), ())),
                            preferred_element_type=jnp.float32)
    gp_ref[...] = _pack(g[:, :HFT], g[:, HFT:])
    up_ref[...] = _pack(u[:, :HFT], u[:, HFT:])
    h = ((g / (1.0 + jnp.exp(-g))) * u).astype(jnp.bfloat16)
    part = jax.lax.dot_general(h, wd_ref[...], (((1,), (1,)), ((), ())),
                               preferred_element_type=jnp.float32)

    @pl.when(f == 0)
    def _():
        out_ref[...] = part
        logits_ref[...] = jax.lax.dot_general(
            xb, wr_ref[...], (((1,), (1,)), ((), ())),
            preferred_element_type=jnp.float32)
        xp_ref[...] = _pack(x[:, :DH], x[:, DH:])

    @pl.when(f != 0)
    def _():
        out_ref[...] += part


def _base_mlp(xf, Wgb, Wub, Wdb, Wrb):
    S = xf.shape[0]
    return pl.pallas_call(
        _k1_body,
        grid=(NFT,),
        in_specs=[
            pl.BlockSpec((S, D), lambda f: (0, 0)),
            pl.BlockSpec((FT, D), lambda f: (f, 0)),
            pl.BlockSpec((FT, D), lambda f: (f, 0)),
            pl.BlockSpec((D, FT), lambda f: (0, f)),
            pl.BlockSpec((E, D), lambda f: (0, 0)),
        ],
        out_specs=[
            pl.BlockSpec((S, HFT), lambda f: (0, f)),
            pl.BlockSpec((S, HFT), lambda f: (0, f)),
            pl.BlockSpec((S, DH), lambda f: (0, 0)),
            pl.BlockSpec((S, D), lambda f: (0, 0)),
            pl.BlockSpec((S, E), lambda f: (0, 0)),
        ],
        out_shape=[
            jax.ShapeDtypeStruct((S, FH), jnp.int32),
            jax.ShapeDtypeStruct((S, FH), jnp.int32),
            jax.ShapeDtypeStruct((S, DH), jnp.int32),
            jax.ShapeDtypeStruct((S, D), jnp.float32),
            jax.ShapeDtypeStruct((S, E), jnp.float32),
        ],
    )(xf, Wgb, Wub, Wdb, Wrb)


def _dispatch(sel, rw):
    S = sel.shape[0]
    onehot = (sel[:, :, None] == jnp.arange(E, dtype=sel.dtype)[None, None, :])
    onehot = onehot.any(axis=1).astype(jnp.int32)
    cumincl = jnp.cumsum(onehot, axis=0)
    counts = cumincl[-1]
    cumexcl = cumincl - onehot
    padded = ((counts + T - 1) // T) * T
    pend = jnp.cumsum(padded)
    poff = (pend - padded).astype(jnp.int32)
    rank = jnp.take_along_axis(cumexcl, sel, axis=1)
    ppos = poff[sel] + rank.astype(jnp.int32)
    tile_expert = jnp.searchsorted(
        pend, jnp.arange(NT, dtype=jnp.int32) * T, side='right')
    tile_expert = jnp.minimum(tile_expert, E - 1).astype(jnp.int32)
    ppos_flat = ppos.reshape(-1)
    slot_w = jnp.zeros((P,), jnp.float32).at[ppos_flat].set(rw.reshape(-1))
    return tile_expert, slot_w, ppos_flat


_K2_CH = 16
_K2_PER_W = NPAIR // NW


def _k2_body(gp_hbm, up_hbm, xp_hbm, tok_hbm, pos_hbm,
             gs_hbm, us_hbm, xs_hbm,
             tk0, tk1, pp0, pp1, g0, g1, u0, u1, x0, x1,
             sg0, sg1, su0, su1, sx0, sx1,
             tg0, tg1, tu0, tu1, tx0, tx1):
    wid = lax.axis_index("s") * NC + lax.axis_index("c")
    base = wid * _K2_PER_W
    n = _K2_PER_W // _K2_CH
    tk = (tk0, tk1)
    pp = (pp0, pp1)
    gb = (g0, g1)
    ub = (u0, u1)
    xb = (x0, x1)
    sg = (sg0, sg1)
    su = (su0, su1)
    sx = (sx0, sx1)
    tg = (tg0, tg1)
    tu = (tu0, tu1)
    tx = (tx0, tx1)

    def start_gather(i):
        par = i % 2
        off = base + i * _K2_CH
        pltpu.sync_copy(tok_hbm.at[pl.ds(off, _K2_CH)], tk[par])
        pltpu.sync_copy(pos_hbm.at[pl.ds(off, _K2_CH)], pp[par])
        cg = pltpu.make_async_copy(gp_hbm.at[tk[par]], gb[par], sg[par])
        cu = pltpu.make_async_copy(up_hbm.at[tk[par]], ub[par], su[par])
        cx = pltpu.make_async_copy(xp_hbm.at[tk[par]], xb[par], sx[par])
        cg.start(); cu.start(); cx.start()
        return cg, cu, cx

    pend_g = start_gather(0)
    pend_s = [None, None]
    for i in range(n):
        par = i % 2
        cg, cu, cx = pend_g
        cg.wait(); cu.wait(); cx.wait()
        wg = pltpu.make_async_copy(gb[par], gs_hbm.at[pp[par]], tg[par])
        wu = pltpu.make_async_copy(ub[par], us_hbm.at[pp[par]], tu[par])
        wx = pltpu.make_async_copy(xb[par], xs_hbm.at[pp[par]], tx[par])
        wg.start(); wu.start(); wx.start()
        pend_s[par] = (wg, wu, wx)
        if i + 1 < n:
            nxt = 1 - par
            if pend_s[nxt] is not None:
                for c in pend_s[nxt]:
                    c.wait()
                pend_s[nxt] = None
            pend_g = start_gather(i + 1)
    for ps in pend_s:
        if ps is not None:
            for c in ps:
                c.wait()


def _stage_gather(gate_p, up_p, x_p, tok_of_pair, ppos_flat):
    mesh = plsc.VectorSubcoreMesh(core_axis_name="c", subcore_axis_name="s")
    return pl.kernel(
        _k2_body,
        out_type=(
            jax.ShapeDtypeStruct((P, FH), jnp.int32),
            jax.ShapeDtypeStruct((P, FH), jnp.int32),
            jax.ShapeDtypeStruct((P, DH), jnp.int32),
        ),
        mesh=mesh,
        scratch_types=[
            pltpu.VMEM((_K2_CH,), jnp.int32),
            pltpu.VMEM((_K2_CH,), jnp.int32),
            pltpu.VMEM((_K2_CH,), jnp.int32),
            pltpu.VMEM((_K2_CH,), jnp.int32),
            pltpu.VMEM((_K2_CH, FH), jnp.int32),
            pltpu.VMEM((_K2_CH, FH), jnp.int32),
            pltpu.VMEM((_K2_CH, FH), jnp.int32),
            pltpu.VMEM((_K2_CH, FH), jnp.int32),
            pltpu.VMEM((_K2_CH, DH), jnp.int32),
            pltpu.VMEM((_K2_CH, DH), jnp.int32),
            pltpu.SemaphoreType.DMA,
            pltpu.SemaphoreType.DMA,
            pltpu.SemaphoreType.DMA,
            pltpu.SemaphoreType.DMA,
            pltpu.SemaphoreType.DMA,
            pltpu.SemaphoreType.DMA,
            pltpu.SemaphoreType.DMA,
            pltpu.SemaphoreType.DMA,
            pltpu.SemaphoreType.DMA,
            pltpu.SemaphoreType.DMA,
            pltpu.SemaphoreType.DMA,
            pltpu.SemaphoreType.DMA,
        ],
    )(gate_p, up_p, x_p, tok_of_pair, ppos_flat)


def _k3_body(te_ref, gs_ref, us_ref, xs_ref, w_ref,
             ag_ref, bgt_ref, au_ref, but_ref, ad_ref, bdt_ref,
             delta_ref):
    xs = _unpack(xs_ref[...]).astype(jnp.bfloat16)
    xag = jax.lax.dot_general(xs, ag_ref[...], (((1,), (1,)), ((), ())),
                              preferred_element_type=jnp.float32)
    gd = jax.lax.dot_general(xag.astype(jnp.bfloat16), bgt_ref[...],
                             (((1,), (0,)), ((), ())),
                             preferred_element_type=jnp.float32)
    xau = jax.lax.dot_general(xs, au_ref[...], (((1,), (1,)), ((), ())),
                              preferred_element_type=jnp.float32)
    ud = jax.lax.dot_general(xau.astype(jnp.bfloat16), but_ref[...],
                             (((1,), (0,)), ((), ())),
                             preferred_element_type=jnp.float32)
    gate = _unpack(gs_ref[...]) + SCALING * gd
    up = _unpack(us_ref[...]) + SCALING * ud
    hidden = ((gate / (1.0 + jnp.exp(-gate))) * up).astype(jnp.bfloat16)
    had = jax.lax.dot_general(hidden, ad_ref[...], (((1,), (1,)), ((), ())),
                              preferred_element_type=jnp.float32)
    had = (had * (SCALING * w_ref[...])).astype(jnp.bfloat16)
    delta_ref[...] = jax.lax.dot_general(
        had, bdt_ref[...], (((1,), (0,)), ((), ())),
        preferred_element_type=jnp.float32)


def _expert_deltas(tile_expert, gs_s, us_s, xs_s, w_col,
                   Agb, BgTb, Aub, BuTb, Adb, BdTb):
    grid_spec = pltpu.PrefetchScalarGridSpec(
        num_scalar_prefetch=1,
        grid=(NT,),
        in_specs=[
            pl.BlockSpec((T, FH), lambda j, te: (j, 0)),
            pl.BlockSpec((T, FH), lambda j, te: (j, 0)),
            pl.BlockSpec((T, DH), lambda j, te: (j, 0)),
            pl.BlockSpec((T, 1), lambda j, te: (j, 0)),
            pl.BlockSpec((None, R, D), lambda j, te: (te[j], 0, 0)),
            pl.BlockSpec((None, R, FF), lambda j, te: (te[j], 0, 0)),
            pl.BlockSpec((None, R, D), lambda j, te: (te[j], 0, 0)),
            pl.BlockSpec((None, R, FF), lambda j, te: (te[j], 0, 0)),
            pl.BlockSpec((None, R, FF), lambda j, te: (te[j], 0, 0)),
            pl.BlockSpec((None, R, D), lambda j, te: (te[j], 0, 0)),
        ],
        out_specs=pl.BlockSpec((T, D), lambda j, te: (j, 0)),
    )
    return pl.pallas_call(
        _k3_body,
        grid_spec=grid_spec,
        out_shape=jax.ShapeDtypeStruct((P, D), jnp.float32),
    )(tile_expert, gs_s, us_s, xs_s, w_col,
      Agb, BgTb, Aub, BuTb, Adb, BdTb)


_K4_CH = 64
_K4_PER_W = NPAIR // NW


def _k4_body(delta_hbm, pos_hbm, dt_hbm, idx_v, rows_v, sem):
    wid = lax.axis_index("s") * NC + lax.axis_index("c")
    base = wid * _K4_PER_W

    def chunk(i, _):
        off = base + i * _K4_CH
        pltpu.sync_copy(pos_hbm.at[pl.ds(off, _K4_CH)], idx_v)
        pltpu.async_copy(delta_hbm.at[idx_v], rows_v, sem).wait()
        pltpu.sync_copy(rows_v, dt_hbm.at[pl.ds(off, _K4_CH)])
        return ()

    lax.fori_loop(0, _K4_PER_W // _K4_CH, chunk, ())


def _delta_to_token_order(delta, ppos_flat):
    mesh = plsc.VectorSubcoreMesh(core_axis_name="c", subcore_axis_name="s")
    return pl.kernel(
        _k4_body,
        out_type=jax.ShapeDtypeStruct((NPAIR, D), jnp.float32),
        mesh=mesh,
        scratch_types=[
            pltpu.VMEM((_K4_CH,), jnp.int32),
            pltpu.VMEM((_K4_CH, D), jnp.float32),
            pltpu.SemaphoreType.DMA,
        ],
    )(delta, ppos_flat)


_K5_T = 128


def _k5_body(dt_ref, base_ref, out_ref):
    d = dt_ref[...].reshape(_K5_T, TOPK, D)
    out_ref[...] = base_ref[...] + d.sum(axis=1)


def _combine(delta_tok, base_out):
    S = base_out.shape[0]
    return pl.pallas_call(
        _k5_body,
        grid=(S // _K5_T,),
        in_specs=[
            pl.BlockSpec((_K5_T * TOPK, D), lambda i: (i, 0)),
            pl.BlockSpec((_K5_T, D), lambda i: (i, 0)),
        ],
        out_specs=pl.BlockSpec((_K5_T, D), lambda i: (i, 0)),
        out_shape=jax.ShapeDtypeStruct((S, D), jnp.float32),
    )(delta_tok, base_out)


def kernel(x, Wg, Wu, Wd, Wr, Ag, Bg, Au, Bu, Ad, Bd):
    b, s, d = x.shape
    xf = x.reshape(-1, d)

    gate_p, up_p, x_p, base_out, logits = _base_mlp(
        xf, Wg.astype(jnp.bfloat16), Wu.astype(jnp.bfloat16),
        Wd.astype(jnp.bfloat16), Wr.astype(jnp.bfloat16))

    probs = jax.nn.softmax(logits, axis=-1)
    rw, sel = jax.lax.top_k(probs, TOPK)
    tile_expert, slot_w, ppos_flat = _dispatch(sel, rw)

    tok_of_pair = jnp.arange(NPAIR, dtype=jnp.int32) // TOPK
    gs_s, us_s, xs_s = _stage_gather(gate_p, up_p, x_p, tok_of_pair,
                                     ppos_flat)

    perm = jnp.asarray(_PERM_FF)
    Agb = Ag.astype(jnp.bfloat16)
    Aub = Au.astype(jnp.bfloat16)
    Adb = Ad[:, :, perm].astype(jnp.bfloat16)
    BgTb = jnp.swapaxes(Bg, 1, 2)[:, :, perm].astype(jnp.bfloat16)
    BuTb = jnp.swapaxes(Bu, 1, 2)[:, :, perm].astype(jnp.bfloat16)
    BdTb = jnp.swapaxes(Bd, 1, 2).astype(jnp.bfloat16)

    delta = _expert_deltas(tile_expert, gs_s, us_s, xs_s,
                           slot_w.reshape(P, 1), Agb, BgTb, Aub, BuTb,
                           Adb, BdTb)

    delta_tok = _delta_to_token_order(delta, ppos_flat)
    return _combine(delta_tok, base_out).reshape(b, s, d)


# speedup vs baseline: 2.0658x; 1.0433x over previous
import functools

import numpy as np
import jax
import jax.numpy as jnp
from jax import lax
from jax.experimental import pallas as pl
from jax.experimental.pallas import tpu as pltpu
from jax.experimental.pallas import tpu_sc as plsc

D = 1024
FF = 2816
E = 64
TOPK = 8
R = 16
SCALING = 2.0

FT = 256
NFT = FF // FT
HFT = FT // 2
FH = FF // 2
DH = D // 2
T = 128
NT = 192
P = NT * T
S_TOK = 2048
NPAIR = S_TOK * TOPK

NC, NS = 2, 16
NW = NC * NS

_PERM_FF = np.concatenate([
    (np.arange(NFT)[:, None] * FT + np.arange(HFT)[None, :]).reshape(-1),
    (np.arange(NFT)[:, None] * FT + HFT + np.arange(HFT)[None, :]).reshape(-1),
])


def _pack(lo, hi):
    lo_u = lax.bitcast_convert_type(
        lo.astype(jnp.bfloat16).astype(jnp.float32), jnp.uint32)
    hi_u = lax.bitcast_convert_type(
        hi.astype(jnp.bfloat16).astype(jnp.float32), jnp.uint32)
    packed = (hi_u & jnp.uint32(0xFFFF0000)) | (lo_u >> 16)
    return lax.bitcast_convert_type(packed, jnp.int32)


def _unpack2(packed_i32):
    u = lax.bitcast_convert_type(packed_i32, jnp.uint32)
    lo = lax.bitcast_convert_type(u << 16, jnp.float32)
    hi = lax.bitcast_convert_type(u & jnp.uint32(0xFFFF0000), jnp.float32)
    return lo, hi


def _unpack(packed_i32):
    lo, hi = _unpack2(packed_i32)
    return jnp.concatenate([lo, hi], axis=1)


def _k1_body(x_ref, wg_ref, wu_ref, wd_ref, wr_ref,
             gp_ref, up_ref, xp_ref, out_ref, logits_ref):
    f = pl.program_id(0)
    x = x_ref[...]
    xb = x.astype(jnp.bfloat16)
    g = jax.lax.dot_general(xb, wg_ref[...], (((1,), (1,)), ((), ())),
                            preferred_element_type=jnp.float32)
    u = jax.lax.dot_general(xb, wu_ref[...], (((1,), (1,)), ((), ())),
                            preferred_element_type=jnp.float32)
    gp_ref[...] = _pack(g[:, :HFT], g[:, HFT:])
    up_ref[...] = _pack(u[:, :HFT], u[:, HFT:])
    h = ((g / (1.0 + jnp.exp(-g))) * u).astype(jnp.bfloat16)
    part = jax.lax.dot_general(h, wd_ref[...], (((1,), (1,)), ((), ())),
                               preferred_element_type=jnp.float32)

    @pl.when(f == 0)
    def _():
        out_ref[...] = part
        logits_ref[...] = jax.lax.dot_general(
            xb, wr_ref[...], (((1,), (1,)), ((), ())),
            preferred_element_type=jnp.float32)
        xp_ref[...] = _pack(x[:, :DH], x[:, DH:])

    @pl.when(f != 0)
    def _():
        out_ref[...] += part


def _base_mlp(xf, Wgb, Wub, Wdb, Wrb):
    S = xf.shape[0]
    return pl.pallas_call(
        _k1_body,
        grid=(NFT,),
        in_specs=[
            pl.BlockSpec((S, D), lambda f: (0, 0)),
            pl.BlockSpec((FT, D), lambda f: (f, 0)),
            pl.BlockSpec((FT, D), lambda f: (f, 0)),
            pl.BlockSpec((D, FT), lambda f: (0, f)),
            pl.BlockSpec((E, D), lambda f: (0, 0)),
        ],
        out_specs=[
            pl.BlockSpec((S, HFT), lambda f: (0, f)),
            pl.BlockSpec((S, HFT), lambda f: (0, f)),
            pl.BlockSpec((S, DH), lambda f: (0, 0)),
            pl.BlockSpec((S, D), lambda f: (0, 0)),
            pl.BlockSpec((S, E), lambda f: (0, 0)),
        ],
        out_shape=[
            jax.ShapeDtypeStruct((S, FH), jnp.int32),
            jax.ShapeDtypeStruct((S, FH), jnp.int32),
            jax.ShapeDtypeStruct((S, DH), jnp.int32),
            jax.ShapeDtypeStruct((S, D), jnp.float32),
            jax.ShapeDtypeStruct((S, E), jnp.float32),
        ],
    )(xf, Wgb, Wub, Wdb, Wrb)


def _dispatch(sel, rw):
    S = sel.shape[0]
    onehot = (sel[:, :, None] == jnp.arange(E, dtype=sel.dtype)[None, None, :])
    onehot = onehot.any(axis=1).astype(jnp.int32)
    cumincl = jnp.cumsum(onehot, axis=0)
    counts = cumincl[-1]
    cumexcl = cumincl - onehot
    padded = ((counts + T - 1) // T) * T
    pend = jnp.cumsum(padded)
    poff = (pend - padded).astype(jnp.int32)
    rank = jnp.take_along_axis(cumexcl, sel, axis=1)
    ppos = poff[sel] + rank.astype(jnp.int32)
    tile_expert = jnp.searchsorted(
        pend, jnp.arange(NT, dtype=jnp.int32) * T, side='right')
    tile_expert = jnp.minimum(tile_expert, E - 1).astype(jnp.int32)
    ppos_flat = ppos.reshape(-1)
    slot_w = jnp.zeros((P,), jnp.float32).at[ppos_flat].set(rw.reshape(-1))
    return tile_expert, slot_w, ppos_flat


_K2_CH = 16
_K2_PER_W = NPAIR // NW


def _k2_body(gp_hbm, up_hbm, xp_hbm, tok_hbm, pos_hbm,
             gs_hbm, us_hbm, xs_hbm,
             tk0, tk1, pp0, pp1, g0, g1, u0, u1, x0, x1,
             sg0, sg1, su0, su1, sx0, sx1,
             tg0, tg1, tu0, tu1, tx0, tx1):
    wid = lax.axis_index("s") * NC + lax.axis_index("c")
    base = wid * _K2_PER_W
    n = _K2_PER_W // _K2_CH
    tk = (tk0, tk1)
    pp = (pp0, pp1)
    gb = (g0, g1)
    ub = (u0, u1)
    xb = (x0, x1)
    sg = (sg0, sg1)
    su = (su0, su1)
    sx = (sx0, sx1)
    tg = (tg0, tg1)
    tu = (tu0, tu1)
    tx = (tx0, tx1)

    def start_gather(i):
        par = i % 2
        off = base + i * _K2_CH
        pltpu.sync_copy(tok_hbm.at[pl.ds(off, _K2_CH)], tk[par])
        pltpu.sync_copy(pos_hbm.at[pl.ds(off, _K2_CH)], pp[par])
        cg = pltpu.make_async_copy(gp_hbm.at[tk[par]], gb[par], sg[par])
        cu = pltpu.make_async_copy(up_hbm.at[tk[par]], ub[par], su[par])
        cx = pltpu.make_async_copy(xp_hbm.at[tk[par]], xb[par], sx[par])
        cg.start(); cu.start(); cx.start()
        return cg, cu, cx

    pend_g = start_gather(0)
    pend_s = [None, None]
    for i in range(n):
        par = i % 2
        cg, cu, cx = pend_g
        cg.wait(); cu.wait(); cx.wait()
        wg = pltpu.make_async_copy(gb[par], gs_hbm.at[pp[par]], tg[par])
        wu = pltpu.make_async_copy(ub[par], us_hbm.at[pp[par]], tu[par])
        wx = pltpu.make_async_copy(xb[par], xs_hbm.at[pp[par]], tx[par])
        wg.start(); wu.start(); wx.start()
        pend_s[par] = (wg, wu, wx)
        if i + 1 < n:
            nxt = 1 - par
            if pend_s[nxt] is not None:
                for c in pend_s[nxt]:
                    c.wait()
                pend_s[nxt] = None
            pend_g = start_gather(i + 1)
    for ps in pend_s:
        if ps is not None:
            for c in ps:
                c.wait()


def _stage_gather(gate_p, up_p, x_p, tok_of_pair, ppos_flat):
    mesh = plsc.VectorSubcoreMesh(core_axis_name="c", subcore_axis_name="s")
    return pl.kernel(
        _k2_body,
        out_type=(
            jax.ShapeDtypeStruct((P, FH), jnp.int32),
            jax.ShapeDtypeStruct((P, FH), jnp.int32),
            jax.ShapeDtypeStruct((P, DH), jnp.int32),
        ),
        mesh=mesh,
        scratch_types=[
            pltpu.VMEM((_K2_CH,), jnp.int32),
            pltpu.VMEM((_K2_CH,), jnp.int32),
            pltpu.VMEM((_K2_CH,), jnp.int32),
            pltpu.VMEM((_K2_CH,), jnp.int32),
            pltpu.VMEM((_K2_CH, FH), jnp.int32),
            pltpu.VMEM((_K2_CH, FH), jnp.int32),
            pltpu.VMEM((_K2_CH, FH), jnp.int32),
            pltpu.VMEM((_K2_CH, FH), jnp.int32),
            pltpu.VMEM((_K2_CH, DH), jnp.int32),
            pltpu.VMEM((_K2_CH, DH), jnp.int32),
            pltpu.SemaphoreType.DMA,
            pltpu.SemaphoreType.DMA,
            pltpu.SemaphoreType.DMA,
            pltpu.SemaphoreType.DMA,
            pltpu.SemaphoreType.DMA,
            pltpu.SemaphoreType.DMA,
            pltpu.SemaphoreType.DMA,
            pltpu.SemaphoreType.DMA,
            pltpu.SemaphoreType.DMA,
            pltpu.SemaphoreType.DMA,
            pltpu.SemaphoreType.DMA,
            pltpu.SemaphoreType.DMA,
        ],
    )(gate_p, up_p, x_p, tok_of_pair, ppos_flat)


def _k3_body(te_ref, gs_ref, us_ref, xs_ref, w_ref,
             ag_ref, bgt_ref, au_ref, but_ref, ad_ref, bdt_ref,
             delta_ref):
    x_lo, x_hi = _unpack2(xs_ref[...])
    xs = jnp.concatenate([x_lo, x_hi], axis=1).astype(jnp.bfloat16)
    xag = jax.lax.dot_general(xs, ag_ref[...], (((1,), (1,)), ((), ())),
                              preferred_element_type=jnp.float32)
    xau = jax.lax.dot_general(xs, au_ref[...], (((1,), (1,)), ((), ())),
                              preferred_element_type=jnp.float32)
    xag = xag.astype(jnp.bfloat16)
    xau = xau.astype(jnp.bfloat16)

    g_lo, g_hi = _unpack2(gs_ref[...])
    u_lo, u_hi = _unpack2(us_ref[...])

    def half(base_g, base_u, bg_h, bu_h):
        gd = jax.lax.dot_general(xag, bg_h, (((1,), (0,)), ((), ())),
                                 preferred_element_type=jnp.float32)
        ud = jax.lax.dot_general(xau, bu_h, (((1,), (0,)), ((), ())),
                                 preferred_element_type=jnp.float32)
        gate = base_g + SCALING * gd
        up = base_u + SCALING * ud
        return ((gate / (1.0 + jnp.exp(-gate))) * up).astype(jnp.bfloat16)

    h_lo = half(g_lo, u_lo, bgt_ref[:, :FH], but_ref[:, :FH])
    h_hi = half(g_hi, u_hi, bgt_ref[:, FH:], but_ref[:, FH:])
    had = jax.lax.dot_general(h_lo, ad_ref[:, :FH], (((1,), (1,)), ((), ())),
                              preferred_element_type=jnp.float32)
    had = had + jax.lax.dot_general(
        h_hi, ad_ref[:, FH:], (((1,), (1,)), ((), ())),
        preferred_element_type=jnp.float32)
    had = (had * (SCALING * w_ref[...])).astype(jnp.bfloat16)
    delta_ref[...] = jax.lax.dot_general(
        had, bdt_ref[...], (((1,), (0,)), ((), ())),
        preferred_element_type=jnp.float32)


def _expert_deltas(tile_expert, gs_s, us_s, xs_s, w_col,
                   Agb, BgTb, Aub, BuTb, Adb, BdTb):
    grid_spec = pltpu.PrefetchScalarGridSpec(
        num_scalar_prefetch=1,
        grid=(NT,),
        in_specs=[
            pl.BlockSpec((T, FH), lambda j, te: (j, 0)),
            pl.BlockSpec((T, FH), lambda j, te: (j, 0)),
            pl.BlockSpec((T, DH), lambda j, te: (j, 0)),
            pl.BlockSpec((T, 1), lambda j, te: (j, 0)),
            pl.BlockSpec((None, R, D), lambda j, te: (te[j], 0, 0)),
            pl.BlockSpec((None, R, FF), lambda j, te: (te[j], 0, 0)),
            pl.BlockSpec((None, R, D), lambda j, te: (te[j], 0, 0)),
            pl.BlockSpec((None, R, FF), lambda j, te: (te[j], 0, 0)),
            pl.BlockSpec((None, R, FF), lambda j, te: (te[j], 0, 0)),
            pl.BlockSpec((None, R, D), lambda j, te: (te[j], 0, 0)),
        ],
        out_specs=pl.BlockSpec((T, D), lambda j, te: (j, 0)),
    )
    return pl.pallas_call(
        _k3_body,
        grid_spec=grid_spec,
        out_shape=jax.ShapeDtypeStruct((P, D), jnp.float32),
    )(tile_expert, gs_s, us_s, xs_s, w_col,
      Agb, BgTb, Aub, BuTb, Adb, BdTb)


_K4_CH = 64
_K4_PER_W = NPAIR // NW


def _k4_body(delta_hbm, pos_hbm, dt_hbm, idx_v, rows_v, sem):
    wid = lax.axis_index("s") * NC + lax.axis_index("c")
    base = wid * _K4_PER_W

    def chunk(i, _):
        off = base + i * _K4_CH
        pltpu.sync_copy(pos_hbm.at[pl.ds(off, _K4_CH)], idx_v)
        pltpu.async_copy(delta_hbm.at[idx_v], rows_v, sem).wait()
        pltpu.sync_copy(rows_v, dt_hbm.at[pl.ds(off, _K4_CH)])
        return ()

    lax.fori_loop(0, _K4_PER_W // _K4_CH, chunk, ())


def _delta_to_token_order(delta, ppos_flat):
    mesh = plsc.VectorSubcoreMesh(core_axis_name="c", subcore_axis_name="s")
    return pl.kernel(
        _k4_body,
        out_type=jax.ShapeDtypeStruct((NPAIR, D), jnp.float32),
        mesh=mesh,
        scratch_types=[
            pltpu.VMEM((_K4_CH,), jnp.int32),
            pltpu.VMEM((_K4_CH, D), jnp.float32),
            pltpu.SemaphoreType.DMA,
        ],
    )(delta, ppos_flat)


_K5_T = 128


def _k5_body(dt_ref, base_ref, out_ref):
    d = dt_ref[...].reshape(_K5_T, TOPK, D)
    out_ref[...] = base_ref[...] + d.sum(axis=1)


def _combine(delta_tok, base_out):
    S = base_out.shape[0]
    return pl.pallas_call(
        _k5_body,
        grid=(S // _K5_T,),
        in_specs=[
            pl.BlockSpec((_K5_T * TOPK, D), lambda i: (i, 0)),
            pl.BlockSpec((_K5_T, D), lambda i: (i, 0)),
        ],
        out_specs=pl.BlockSpec((_K5_T, D), lambda i: (i, 0)),
        out_shape=jax.ShapeDtypeStruct((S, D), jnp.float32),
    )(delta_tok, base_out)


def kernel(x, Wg, Wu, Wd, Wr, Ag, Bg, Au, Bu, Ad, Bd):
    b, s, d = x.shape
    xf = x.reshape(-1, d)

    gate_p, up_p, x_p, base_out, logits = _base_mlp(
        xf, Wg.astype(jnp.bfloat16), Wu.astype(jnp.bfloat16),
        Wd.astype(jnp.bfloat16), Wr.astype(jnp.bfloat16))

    probs = jax.nn.softmax(logits, axis=-1)
    rw, sel = jax.lax.top_k(probs, TOPK)
    tile_expert, slot_w, ppos_flat = _dispatch(sel, rw)

    tok_of_pair = jnp.arange(NPAIR, dtype=jnp.int32) // TOPK
    gs_s, us_s, xs_s = _stage_gather(gate_p, up_p, x_p, tok_of_pair,
                                     ppos_flat)

    perm = jnp.asarray(_PERM_FF)
    Agb = Ag.astype(jnp.bfloat16)
    Aub = Au.astype(jnp.bfloat16)
    Adb = Ad[:, :, perm].astype(jnp.bfloat16)
    BgTb = jnp.swapaxes(Bg, 1, 2)[:, :, perm].astype(jnp.bfloat16)
    BuTb = jnp.swapaxes(Bu, 1, 2)[:, :, perm].astype(jnp.bfloat16)
    BdTb = jnp.swapaxes(Bd, 1, 2).astype(jnp.bfloat16)

    delta = _expert_deltas(tile_expert, gs_s, us_s, xs_s,
                           slot_w.reshape(P, 1), Agb, BgTb, Aub, BuTb,
                           Adb, BdTb)

    delta_tok = _delta_to_token_order(delta, ppos_flat)
    return _combine(delta_tok, base_out).reshape(b, s, d)


# speedup vs baseline: 2.0764x; 1.0051x over previous
import functools

import numpy as np
import jax
import jax.numpy as jnp
from jax import lax
from jax.experimental import pallas as pl
from jax.experimental.pallas import tpu as pltpu
from jax.experimental.pallas import tpu_sc as plsc

D = 1024
FF = 2816
E = 64
TOPK = 8
R = 16
SCALING = 2.0

FT = 256
NFT = FF // FT
HFT = FT // 2
FH = FF // 2
DH = D // 2
T = 128
NT = 192
P = NT * T
S_TOK = 2048
NPAIR = S_TOK * TOPK

NC, NS = 2, 16
NW = NC * NS

_PERM_FF = np.concatenate([
    (np.arange(NFT)[:, None] * FT + np.arange(HFT)[None, :]).reshape(-1),
    (np.arange(NFT)[:, None] * FT + HFT + np.arange(HFT)[None, :]).reshape(-1),
])


def _pack(lo, hi):
    lo_u = lax.bitcast_convert_type(
        lo.astype(jnp.bfloat16).astype(jnp.float32), jnp.uint32)
    hi_u = lax.bitcast_convert_type(
        hi.astype(jnp.bfloat16).astype(jnp.float32), jnp.uint32)
    packed = (hi_u & jnp.uint32(0xFFFF0000)) | (lo_u >> 16)
    return lax.bitcast_convert_type(packed, jnp.int32)


def _unpack2(packed_i32):
    u = lax.bitcast_convert_type(packed_i32, jnp.uint32)
    lo = lax.bitcast_convert_type(u << 16, jnp.float32)
    hi = lax.bitcast_convert_type(u & jnp.uint32(0xFFFF0000), jnp.float32)
    return lo, hi


def _unpack(packed_i32):
    lo, hi = _unpack2(packed_i32)
    return jnp.concatenate([lo, hi], axis=1)


def _k1_body(x_ref, wg_ref, wu_ref, wd_ref, wr_ref,
             gp_ref, up_ref, xp_ref, out_ref, logits_ref):
    f = pl.program_id(0)
    x = x_ref[...]
    xb = x.astype(jnp.bfloat16)
    g = jax.lax.dot_general(xb, wg_ref[...], (((1,), (1,)), ((), ())),
                            preferred_element_type=jnp.float32)
    u = jax.lax.dot_general(xb, wu_ref[...], (((1,), (1,)), ((), ())),
                            preferred_element_type=jnp.float32)
    gp_ref[...] = _pack(g[:, :HFT], g[:, HFT:])
    up_ref[...] = _pack(u[:, :HFT], u[:, HFT:])
    h = ((g / (1.0 + jnp.exp(-g))) * u).astype(jnp.bfloat16)
    part = jax.lax.dot_general(h, wd_ref[...], (((1,), (1,)), ((), ())),
                               preferred_element_type=jnp.float32)

    @pl.when(f == 0)
    def _():
        out_ref[...] = part
        logits_ref[...] = jax.lax.dot_general(
            xb, wr_ref[...], (((1,), (1,)), ((), ())),
            preferred_element_type=jnp.float32)
        xp_ref[...] = _pack(x[:, :DH], x[:, DH:])

    @pl.when(f != 0)
    def _():
        out_ref[...] += part


def _base_mlp(xf, Wgb, Wub, Wdb, Wrb):
    S = xf.shape[0]
    return pl.pallas_call(
        _k1_body,
        grid=(NFT,),
        in_specs=[
            pl.BlockSpec((S, D), lambda f: (0, 0)),
            pl.BlockSpec((FT, D), lambda f: (f, 0)),
            pl.BlockSpec((FT, D), lambda f: (f, 0)),
            pl.BlockSpec((D, FT), lambda f: (0, f)),
            pl.BlockSpec((E, D), lambda f: (0, 0)),
        ],
        out_specs=[
            pl.BlockSpec((S, HFT), lambda f: (0, f)),
            pl.BlockSpec((S, HFT), lambda f: (0, f)),
            pl.BlockSpec((S, DH), lambda f: (0, 0)),
            pl.BlockSpec((S, D), lambda f: (0, 0)),
            pl.BlockSpec((S, E), lambda f: (0, 0)),
        ],
        out_shape=[
            jax.ShapeDtypeStruct((S, FH), jnp.int32),
            jax.ShapeDtypeStruct((S, FH), jnp.int32),
            jax.ShapeDtypeStruct((S, DH), jnp.int32),
            jax.ShapeDtypeStruct((S, D), jnp.float32),
            jax.ShapeDtypeStruct((S, E), jnp.float32),
        ],
    )(xf, Wgb, Wub, Wdb, Wrb)


def _dispatch(sel, rw):
    S = sel.shape[0]
    onehot = (sel[:, :, None] == jnp.arange(E, dtype=sel.dtype)[None, None, :])
    onehot = onehot.any(axis=1).astype(jnp.int32)
    cumincl = jnp.cumsum(onehot, axis=0)
    counts = cumincl[-1]
    cumexcl = cumincl - onehot
    padded = ((counts + T - 1) // T) * T
    pend = jnp.cumsum(padded)
    poff = (pend - padded).astype(jnp.int32)
    rank = jnp.take_along_axis(cumexcl, sel, axis=1)
    ppos = poff[sel] + rank.astype(jnp.int32)
    tile_expert = jnp.searchsorted(
        pend, jnp.arange(NT, dtype=jnp.int32) * T, side='right')
    tile_expert = jnp.minimum(tile_expert, E - 1).astype(jnp.int32)
    ppos_flat = ppos.reshape(-1)
    slot_w = jnp.zeros((P,), jnp.float32).at[ppos_flat].set(rw.reshape(-1))
    return tile_expert, slot_w, ppos_flat


_K2_CH = 16
_K2_PER_W = NPAIR // NW


def _k2_body(gp_hbm, up_hbm, xp_hbm, tok_hbm, pos_hbm,
             gs_hbm, us_hbm, xs_hbm,
             tk0, tk1, pp0, pp1, g0, g1, u0, u1, x0, x1,
             sg0, sg1, su0, su1, sx0, sx1,
             tg0, tg1, tu0, tu1, tx0, tx1):
    wid = lax.axis_index("s") * NC + lax.axis_index("c")
    base = wid * _K2_PER_W
    n = _K2_PER_W // _K2_CH
    tk = (tk0, tk1)
    pp = (pp0, pp1)
    gb = (g0, g1)
    ub = (u0, u1)
    xb = (x0, x1)
    sg = (sg0, sg1)
    su = (su0, su1)
    sx = (sx0, sx1)
    tg = (tg0, tg1)
    tu = (tu0, tu1)
    tx = (tx0, tx1)

    def start_gather(i):
        par = i % 2
        off = base + i * _K2_CH
        pltpu.sync_copy(tok_hbm.at[pl.ds(off, _K2_CH)], tk[par])
        pltpu.sync_copy(pos_hbm.at[pl.ds(off, _K2_CH)], pp[par])
        cg = pltpu.make_async_copy(gp_hbm.at[tk[par]], gb[par], sg[par])
        cu = pltpu.make_async_copy(up_hbm.at[tk[par]], ub[par], su[par])
        cx = pltpu.make_async_copy(xp_hbm.at[tk[par]], xb[par], sx[par])
        cg.start(); cu.start(); cx.start()
        return cg, cu, cx

    pend_g = start_gather(0)
    pend_s = [None, None]
    for i in range(n):
        par = i % 2
        cg, cu, cx = pend_g
        cg.wait(); cu.wait(); cx.wait()
        wg = pltpu.make_async_copy(gb[par], gs_hbm.at[pp[par]], tg[par])
        wu = pltpu.make_async_copy(ub[par], us_hbm.at[pp[par]], tu[par])
        wx = pltpu.make_async_copy(xb[par], xs_hbm.at[pp[par]], tx[par])
        wg.start(); wu.start(); wx.start()
        pend_s[par] = (wg, wu, wx)
        if i + 1 < n:
            nxt = 1 - par
            if pend_s[nxt] is not None:
                for c in pend_s[nxt]:
                    c.wait()
                pend_s[nxt] = None
            pend_g = start_gather(i + 1)
    for ps in pend_s:
        if ps is not None:
            for c in ps:
                c.wait()


def _stage_gather(gate_p, up_p, x_p, tok_of_pair, ppos_flat):
    mesh = plsc.VectorSubcoreMesh(core_axis_name="c", subcore_axis_name="s")
    return pl.kernel(
        _k2_body,
        out_type=(
            jax.ShapeDtypeStruct((P, FH), jnp.int32),
            jax.ShapeDtypeStruct((P, FH), jnp.int32),
            jax.ShapeDtypeStruct((P, DH), jnp.int32),
        ),
        mesh=mesh,
        scratch_types=[
            pltpu.VMEM((_K2_CH,), jnp.int32),
            pltpu.VMEM((_K2_CH,), jnp.int32),
            pltpu.VMEM((_K2_CH,), jnp.int32),
            pltpu.VMEM((_K2_CH,), jnp.int32),
            pltpu.VMEM((_K2_CH, FH), jnp.int32),
            pltpu.VMEM((_K2_CH, FH), jnp.int32),
            pltpu.VMEM((_K2_CH, FH), jnp.int32),
            pltpu.VMEM((_K2_CH, FH), jnp.int32),
            pltpu.VMEM((_K2_CH, DH), jnp.int32),
            pltpu.VMEM((_K2_CH, DH), jnp.int32),
            pltpu.SemaphoreType.DMA,
            pltpu.SemaphoreType.DMA,
            pltpu.SemaphoreType.DMA,
            pltpu.SemaphoreType.DMA,
            pltpu.SemaphoreType.DMA,
            pltpu.SemaphoreType.DMA,
            pltpu.SemaphoreType.DMA,
            pltpu.SemaphoreType.DMA,
            pltpu.SemaphoreType.DMA,
            pltpu.SemaphoreType.DMA,
            pltpu.SemaphoreType.DMA,
            pltpu.SemaphoreType.DMA,
        ],
    )(gate_p, up_p, x_p, tok_of_pair, ppos_flat)


def _k3_body(te_ref, gs_ref, us_ref, xs_ref, w_ref,
             ag_ref, bgt_ref, but_ref, ad_ref, bdt_ref,
             delta_ref):
    x_lo, x_hi = _unpack2(xs_ref[...])
    xa = jax.lax.dot_general(
        x_lo.astype(jnp.bfloat16), ag_ref[:, :DH], (((1,), (1,)), ((), ())),
        preferred_element_type=jnp.float32)
    xa = xa + jax.lax.dot_general(
        x_hi.astype(jnp.bfloat16), ag_ref[:, DH:], (((1,), (1,)), ((), ())),
        preferred_element_type=jnp.float32)
    xa = xa.astype(jnp.bfloat16)
    xag = xa[:, :R]
    xau = xa[:, R:]

    g_lo, g_hi = _unpack2(gs_ref[...])
    u_lo, u_hi = _unpack2(us_ref[...])

    def half(base_g, base_u, bg_h, bu_h):
        gd = jax.lax.dot_general(xag, bg_h, (((1,), (0,)), ((), ())),
                                 preferred_element_type=jnp.float32)
        ud = jax.lax.dot_general(xau, bu_h, (((1,), (0,)), ((), ())),
                                 preferred_element_type=jnp.float32)
        gate = base_g + SCALING * gd
        up = base_u + SCALING * ud
        return ((gate / (1.0 + jnp.exp(-gate))) * up).astype(jnp.bfloat16)

    h_lo = half(g_lo, u_lo, bgt_ref[:, :FH], but_ref[:, :FH])
    h_hi = half(g_hi, u_hi, bgt_ref[:, FH:], but_ref[:, FH:])
    had = jax.lax.dot_general(h_lo, ad_ref[:, :FH], (((1,), (1,)), ((), ())),
                              preferred_element_type=jnp.float32)
    had = had + jax.lax.dot_general(
        h_hi, ad_ref[:, FH:], (((1,), (1,)), ((), ())),
        preferred_element_type=jnp.float32)
    had = (had * (SCALING * w_ref[...])).astype(jnp.bfloat16)
    delta_ref[...] = jax.lax.dot_general(
        had, bdt_ref[...], (((1,), (0,)), ((), ())),
        preferred_element_type=jnp.float32)


def _expert_deltas(tile_expert, gs_s, us_s, xs_s, w_col,
                   AgAub, BgTb, BuTb, Adb, BdTb):
    grid_spec = pltpu.PrefetchScalarGridSpec(
        num_scalar_prefetch=1,
        grid=(NT,),
        in_specs=[
            pl.BlockSpec((T, FH), lambda j, te: (j, 0)),
            pl.BlockSpec((T, FH), lambda j, te: (j, 0)),
            pl.BlockSpec((T, DH), lambda j, te: (j, 0)),
            pl.BlockSpec((T, 1), lambda j, te: (j, 0)),
            pl.BlockSpec((None, 2 * R, D), lambda j, te: (te[j], 0, 0)),
            pl.BlockSpec((None, R, FF), lambda j, te: (te[j], 0, 0)),
            pl.BlockSpec((None, R, FF), lambda j, te: (te[j], 0, 0)),
            pl.BlockSpec((None, R, FF), lambda j, te: (te[j], 0, 0)),
            pl.BlockSpec((None, R, D), lambda j, te: (te[j], 0, 0)),
        ],
        out_specs=pl.BlockSpec((T, D), lambda j, te: (j, 0)),
    )
    return pl.pallas_call(
        _k3_body,
        grid_spec=grid_spec,
        out_shape=jax.ShapeDtypeStruct((P, D), jnp.float32),
    )(tile_expert, gs_s, us_s, xs_s, w_col,
      AgAub, BgTb, BuTb, Adb, BdTb)


_K4_CH = 64
_K4_PER_W = NPAIR // NW


def _k4_body(delta_hbm, pos_hbm, dt_hbm, idx_v, rows_v, sem):
    wid = lax.axis_index("s") * NC + lax.axis_index("c")
    base = wid * _K4_PER_W

    def chunk(i, _):
        off = base + i * _K4_CH
        pltpu.sync_copy(pos_hbm.at[pl.ds(off, _K4_CH)], idx_v)
        pltpu.async_copy(delta_hbm.at[idx_v], rows_v, sem).wait()
        pltpu.sync_copy(rows_v, dt_hbm.at[pl.ds(off, _K4_CH)])
        return ()

    lax.fori_loop(0, _K4_PER_W // _K4_CH, chunk, ())


def _delta_to_token_order(delta, ppos_flat):
    mesh = plsc.VectorSubcoreMesh(core_axis_name="c", subcore_axis_name="s")
    return pl.kernel(
        _k4_body,
        out_type=jax.ShapeDtypeStruct((NPAIR, D), jnp.float32),
        mesh=mesh,
        scratch_types=[
            pltpu.VMEM((_K4_CH,), jnp.int32),
            pltpu.VMEM((_K4_CH, D), jnp.float32),
            pltpu.SemaphoreType.DMA,
        ],
    )(delta, ppos_flat)


_K5_T = 128


def _k5_body(dt_ref, base_ref, out_ref):
    d = dt_ref[...].reshape(_K5_T, TOPK, D)
    out_ref[...] = base_ref[...] + d.sum(axis=1)


def _combine(delta_tok, base_out):
    S = base_out.shape[0]
    return pl.pallas_call(
        _k5_body,
        grid=(S // _K5_T,),
        in_specs=[
            pl.BlockSpec((_K5_T * TOPK, D), lambda i: (i, 0)),
            pl.BlockSpec((_K5_T, D), lambda i: (i, 0)),
        ],
        out_specs=pl.BlockSpec((_K5_T, D), lambda i: (i, 0)),
        out_shape=jax.ShapeDtypeStruct((S, D), jnp.float32),
    )(delta_tok, base_out)


def kernel(x, Wg, Wu, Wd, Wr, Ag, Bg, Au, Bu, Ad, Bd):
    b, s, d = x.shape
    xf = x.reshape(-1, d)

    gate_p, up_p, x_p, base_out, logits = _base_mlp(
        xf, Wg.astype(jnp.bfloat16), Wu.astype(jnp.bfloat16),
        Wd.astype(jnp.bfloat16), Wr.astype(jnp.bfloat16))

    probs = jax.nn.softmax(logits, axis=-1)
    rw, sel = jax.lax.top_k(probs, TOPK)
    tile_expert, slot_w, ppos_flat = _dispatch(sel, rw)

    tok_of_pair = jnp.arange(NPAIR, dtype=jnp.int32) // TOPK
    gs_s, us_s, xs_s = _stage_gather(gate_p, up_p, x_p, tok_of_pair,
                                     ppos_flat)

    perm = jnp.asarray(_PERM_FF)
    AgAub = jnp.concatenate([Ag, Au], axis=1).astype(jnp.bfloat16)
    Adb = Ad[:, :, perm].astype(jnp.bfloat16)
    BgTb = jnp.swapaxes(Bg, 1, 2)[:, :, perm].astype(jnp.bfloat16)
    BuTb = jnp.swapaxes(Bu, 1, 2)[:, :, perm].astype(jnp.bfloat16)
    BdTb = jnp.swapaxes(Bd, 1, 2).astype(jnp.bfloat16)

    delta = _expert_deltas(tile_expert, gs_s, us_s, xs_s,
                           slot_w.reshape(P, 1), AgAub, BgTb, BuTb,
                           Adb, BdTb)

    delta_tok = _delta_to_token_order(delta, ppos_flat)
    return _combine(delta_tok, base_out).reshape(b, s, d)


# speedup vs baseline: 2.1185x; 1.0203x over previous
import numpy as np
import jax
import jax.numpy as jnp
from jax import lax
from jax.experimental import pallas as pl
from jax.experimental.pallas import tpu as pltpu
from jax.experimental.pallas import tpu_sc as plsc

D = 1024
FF = 2816
E = 64
TOPK = 8
R = 16
SCALING = 2.0

FT = 256
NFT = FF // FT
HFT = FT // 2
FH = FF // 2
DH = D // 2
T = 128
NT = 192
P = NT * T
S_TOK = 2048
NPAIR = S_TOK * TOPK

NC, NS = 2, 16
NW = NC * NS

_PERM_FF = np.concatenate([
    (np.arange(NFT)[:, None] * FT + np.arange(HFT)[None, :]).reshape(-1),
    (np.arange(NFT)[:, None] * FT + HFT + np.arange(HFT)[None, :]).reshape(-1),
])


def _pack(lo, hi):
    lo_u = lax.bitcast_convert_type(
        lo.astype(jnp.bfloat16).astype(jnp.float32), jnp.uint32)
    hi_u = lax.bitcast_convert_type(
        hi.astype(jnp.bfloat16).astype(jnp.float32), jnp.uint32)
    packed = (hi_u & jnp.uint32(0xFFFF0000)) | (lo_u >> 16)
    return lax.bitcast_convert_type(packed, jnp.int32)


def _unpack2(packed_i32):
    u = lax.bitcast_convert_type(packed_i32, jnp.uint32)
    lo = lax.bitcast_convert_type(u << 16, jnp.float32)
    hi = lax.bitcast_convert_type(u & jnp.uint32(0xFFFF0000), jnp.float32)
    return lo, hi


def _k1_body(x_ref, wg_ref, wu_ref, wd_ref, wr_ref,
             gp_ref, up_ref, xp_ref, out_ref, logits_ref):
    f = pl.program_id(0)
    x = x_ref[...]
    xb = x.astype(jnp.bfloat16)
    g = jax.lax.dot_general(xb, wg_ref[...], (((1,), (1,)), ((), ())),
                            preferred_element_type=jnp.float32)
    u = jax.lax.dot_general(xb, wu_ref[...], (((1,), (1,)), ((), ())),
                            preferred_element_type=jnp.float32)
    gp_ref[...] = _pack(g[:, :HFT], g[:, HFT:])
    up_ref[...] = _pack(u[:, :HFT], u[:, HFT:])
    h = ((g / (1.0 + jnp.exp(-g))) * u).astype(jnp.bfloat16)
    part = jax.lax.dot_general(h, wd_ref[...], (((1,), (1,)), ((), ())),
                               preferred_element_type=jnp.float32)

    @pl.when(f == 0)
    def _():
        out_ref[...] = part
        logits_ref[...] = jax.lax.dot_general(
            xb, wr_ref[...], (((1,), (1,)), ((), ())),
            preferred_element_type=jnp.float32)
        xp_ref[...] = _pack(x[:, :DH], x[:, DH:])

    @pl.when(f != 0)
    def _():
        out_ref[...] += part


def _base_mlp(xf, Wgb, Wub, Wdb, Wrb):
    S = xf.shape[0]
    return pl.pallas_call(
        _k1_body,
        grid=(NFT,),
        in_specs=[
            pl.BlockSpec((S, D), lambda f: (0, 0)),
            pl.BlockSpec((FT, D), lambda f: (f, 0)),
            pl.BlockSpec((FT, D), lambda f: (f, 0)),
            pl.BlockSpec((D, FT), lambda f: (0, f)),
            pl.BlockSpec((E, D), lambda f: (0, 0)),
        ],
        out_specs=[
            pl.BlockSpec((S, HFT), lambda f: (0, f)),
            pl.BlockSpec((S, HFT), lambda f: (0, f)),
            pl.BlockSpec((S, DH), lambda f: (0, 0)),
            pl.BlockSpec((S, D), lambda f: (0, 0)),
            pl.BlockSpec((S, E), lambda f: (0, 0)),
        ],
        out_shape=[
            jax.ShapeDtypeStruct((S, FH), jnp.int32),
            jax.ShapeDtypeStruct((S, FH), jnp.int32),
            jax.ShapeDtypeStruct((S, DH), jnp.int32),
            jax.ShapeDtypeStruct((S, D), jnp.float32),
            jax.ShapeDtypeStruct((S, E), jnp.float32),
        ],
    )(xf, Wgb, Wub, Wdb, Wrb)


def _dispatch(sel, rw):
    S = sel.shape[0]
    onehot = (sel[:, :, None] == jnp.arange(E, dtype=sel.dtype)[None, None, :])
    onehot = onehot.any(axis=1).astype(jnp.int32)
    cumincl = jnp.cumsum(onehot, axis=0)
    counts = cumincl[-1]
    cumexcl = cumincl - onehot
    padded = ((counts + T - 1) // T) * T
    pend = jnp.cumsum(padded)
    poff = (pend - padded).astype(jnp.int32)
    rank = jnp.take_along_axis(cumexcl, sel, axis=1)
    ppos = poff[sel] + rank.astype(jnp.int32)
    tile_expert = jnp.searchsorted(
        pend, jnp.arange(NT, dtype=jnp.int32) * T, side='right')
    tile_expert = jnp.minimum(tile_expert, E - 1).astype(jnp.int32)
    ppos_flat = ppos.reshape(-1)
    slot_w = jnp.zeros((P,), jnp.float32).at[ppos_flat].set(rw.reshape(-1))
    return tile_expert, slot_w, ppos_flat


_K2_CH = 16
_K2_PER_W = NPAIR // NW


def _k2_body(gp_hbm, up_hbm, xp_hbm, tok_hbm, pos_hbm,
             gs_hbm, us_hbm, xs_hbm,
             tk0, tk1, pp0, pp1, g0, g1, u0, u1, x0, x1,
             sg0, sg1, su0, su1, sx0, sx1,
             tg0, tg1, tu0, tu1, tx0, tx1):
    wid = lax.axis_index("s") * NC + lax.axis_index("c")
    base = wid * _K2_PER_W
    n = _K2_PER_W // _K2_CH
    tk = (tk0, tk1)
    pp = (pp0, pp1)
    gb = (g0, g1)
    ub = (u0, u1)
    xb = (x0, x1)
    sg = (sg0, sg1)
    su = (su0, su1)
    sx = (sx0, sx1)
    tg = (tg0, tg1)
    tu = (tu0, tu1)
    tx = (tx0, tx1)

    def start_gather(i):
        par = i % 2
        off = base + i * _K2_CH
        pltpu.sync_copy(tok_hbm.at[pl.ds(off, _K2_CH)], tk[par])
        pltpu.sync_copy(pos_hbm.at[pl.ds(off, _K2_CH)], pp[par])
        cg = pltpu.make_async_copy(gp_hbm.at[tk[par]], gb[par], sg[par])
        cu = pltpu.make_async_copy(up_hbm.at[tk[par]], ub[par], su[par])
        cx = pltpu.make_async_copy(xp_hbm.at[tk[par]], xb[par], sx[par])
        cg.start(); cu.start(); cx.start()
        return cg, cu, cx

    pend_g = start_gather(0)
    pend_s = [None, None]
    for i in range(n):
        par = i % 2
        cg, cu, cx = pend_g
        cg.wait(); cu.wait(); cx.wait()
        wg = pltpu.make_async_copy(gb[par], gs_hbm.at[pp[par]], tg[par])
        wu = pltpu.make_async_copy(ub[par], us_hbm.at[pp[par]], tu[par])
        wx = pltpu.make_async_copy(xb[par], xs_hbm.at[pp[par]], tx[par])
        wg.start(); wu.start(); wx.start()
        pend_s[par] = (wg, wu, wx)
        if i + 1 < n:
            nxt = 1 - par
            if pend_s[nxt] is not None:
                for c in pend_s[nxt]:
                    c.wait()
                pend_s[nxt] = None
            pend_g = start_gather(i + 1)
    for ps in pend_s:
        if ps is not None:
            for c in ps:
                c.wait()


def _stage_gather(gate_p, up_p, x_p, tok_of_pair, ppos_flat):
    mesh = plsc.VectorSubcoreMesh(core_axis_name="c", subcore_axis_name="s")
    return pl.kernel(
        _k2_body,
        out_type=(
            jax.ShapeDtypeStruct((P, FH), jnp.int32),
            jax.ShapeDtypeStruct((P, FH), jnp.int32),
            jax.ShapeDtypeStruct((P, DH), jnp.int32),
        ),
        mesh=mesh,
        scratch_types=[
            pltpu.VMEM((_K2_CH,), jnp.int32),
            pltpu.VMEM((_K2_CH,), jnp.int32),
            pltpu.VMEM((_K2_CH,), jnp.int32),
            pltpu.VMEM((_K2_CH,), jnp.int32),
            pltpu.VMEM((_K2_CH, FH), jnp.int32),
            pltpu.VMEM((_K2_CH, FH), jnp.int32),
            pltpu.VMEM((_K2_CH, FH), jnp.int32),
            pltpu.VMEM((_K2_CH, FH), jnp.int32),
            pltpu.VMEM((_K2_CH, DH), jnp.int32),
            pltpu.VMEM((_K2_CH, DH), jnp.int32),
            pltpu.SemaphoreType.DMA,
            pltpu.SemaphoreType.DMA,
            pltpu.SemaphoreType.DMA,
            pltpu.SemaphoreType.DMA,
            pltpu.SemaphoreType.DMA,
            pltpu.SemaphoreType.DMA,
            pltpu.SemaphoreType.DMA,
            pltpu.SemaphoreType.DMA,
            pltpu.SemaphoreType.DMA,
            pltpu.SemaphoreType.DMA,
            pltpu.SemaphoreType.DMA,
            pltpu.SemaphoreType.DMA,
        ],
    )(gate_p, up_p, x_p, tok_of_pair, ppos_flat)


def _k3_body(te_ref, gs_ref, us_ref, xs_ref, w_ref,
             ag_ref, bgt_ref, but_ref, ad_ref, bdt_ref,
             delta_ref):
    x_lo, x_hi = _unpack2(xs_ref[...])
    xa = jax.lax.dot_general(
        x_lo.astype(jnp.bfloat16), ag_ref[:, :DH], (((1,), (1,)), ((), ())),
        preferred_element_type=jnp.float32)
    xa = xa + jax.lax.dot_general(
        x_hi.astype(jnp.bfloat16), ag_ref[:, DH:], (((1,), (1,)), ((), ())),
        preferred_element_type=jnp.float32)
    xa = xa.astype(jnp.bfloat16)
    xag = xa[:, :R]
    xau = xa[:, R:]

    g_lo, g_hi = _unpack2(gs_ref[...])
    u_lo, u_hi = _unpack2(us_ref[...])

    def half(base_g, base_u, bg_h, bu_h):
        gd = jax.lax.dot_general(xag, bg_h, (((1,), (0,)), ((), ())),
                                 preferred_element_type=jnp.float32)
        ud = jax.lax.dot_general(xau, bu_h, (((1,), (0,)), ((), ())),
                                 preferred_element_type=jnp.float32)
        gate = base_g + SCALING * gd
        up = base_u + SCALING * ud
        return ((gate / (1.0 + jnp.exp(-gate))) * up).astype(jnp.bfloat16)

    h_lo = half(g_lo, u_lo, bgt_ref[:, :FH], but_ref[:, :FH])
    h_hi = half(g_hi, u_hi, bgt_ref[:, FH:], but_ref[:, FH:])
    had = jax.lax.dot_general(h_lo, ad_ref[:, :FH], (((1,), (1,)), ((), ())),
                              preferred_element_type=jnp.float32)
    had = had + jax.lax.dot_general(
        h_hi, ad_ref[:, FH:], (((1,), (1,)), ((), ())),
        preferred_element_type=jnp.float32)
    had = (had * (SCALING * w_ref[...])).astype(jnp.bfloat16)
    delta_ref[...] = jax.lax.dot_general(
        had, bdt_ref[...], (((1,), (0,)), ((), ())),
        preferred_element_type=jnp.float32)


def _expert_deltas(tile_expert, gs_s, us_s, xs_s, w_col,
                   AgAub, BgTb, BuTb, Adb, BdTb):
    grid_spec = pltpu.PrefetchScalarGridSpec(
        num_scalar_prefetch=1,
        grid=(NT,),
        in_specs=[
            pl.BlockSpec((T, FH), lambda j, te: (j, 0)),
            pl.BlockSpec((T, FH), lambda j, te: (j, 0)),
            pl.BlockSpec((T, DH), lambda j, te: (j, 0)),
            pl.BlockSpec((T, 1), lambda j, te: (j, 0)),
            pl.BlockSpec((None, 2 * R, D), lambda j, te: (te[j], 0, 0)),
            pl.BlockSpec((None, R, FF), lambda j, te: (te[j], 0, 0)),
            pl.BlockSpec((None, R, FF), lambda j, te: (te[j], 0, 0)),
            pl.BlockSpec((None, R, FF), lambda j, te: (te[j], 0, 0)),
            pl.BlockSpec((None, R, D), lambda j, te: (te[j], 0, 0)),
        ],
        out_specs=pl.BlockSpec((T, D), lambda j, te: (j, 0)),
    )
    return pl.pallas_call(
        _k3_body,
        grid_spec=grid_spec,
        out_shape=jax.ShapeDtypeStruct((P, D), jnp.float32),
    )(tile_expert, gs_s, us_s, xs_s, w_col,
      AgAub, BgTb, BuTb, Adb, BdTb)


_K4_CH = 64
_K4_PER_W = NPAIR // NW


def _k4_body(delta_hbm, pos_hbm, dt_hbm, idx_v, rows_v, sem):
    wid = lax.axis_index("s") * NC + lax.axis_index("c")
    base = wid * _K4_PER_W

    def chunk(i, _):
        off = base + i * _K4_CH
        pltpu.sync_copy(pos_hbm.at[pl.ds(off, _K4_CH)], idx_v)
        pltpu.async_copy(delta_hbm.at[idx_v], rows_v, sem).wait()
        pltpu.sync_copy(rows_v, dt_hbm.at[pl.ds(off, _K4_CH)])
        return ()

    lax.fori_loop(0, _K4_PER_W // _K4_CH, chunk, ())


def _delta_to_token_order(delta, ppos_flat):
    mesh = plsc.VectorSubcoreMesh(core_axis_name="c", subcore_axis_name="s")
    return pl.kernel(
        _k4_body,
        out_type=jax.ShapeDtypeStruct((NPAIR, D), jnp.float32),
        mesh=mesh,
        scratch_types=[
            pltpu.VMEM((_K4_CH,), jnp.int32),
            pltpu.VMEM((_K4_CH, D), jnp.float32),
            pltpu.SemaphoreType.DMA,
        ],
    )(delta, ppos_flat)


_K5_T = 128


def _k5_body(dt_ref, base_ref, out_ref):
    d = dt_ref[...].reshape(_K5_T, TOPK, D)
    out_ref[...] = base_ref[...] + d.sum(axis=1)


def _combine(delta_tok, base_out):
    S = base_out.shape[0]
    return pl.pallas_call(
        _k5_body,
        grid=(S // _K5_T,),
        in_specs=[
            pl.BlockSpec((_K5_T * TOPK, D), lambda i: (i, 0)),
            pl.BlockSpec((_K5_T, D), lambda i: (i, 0)),
        ],
        out_specs=pl.BlockSpec((_K5_T, D), lambda i: (i, 0)),
        out_shape=jax.ShapeDtypeStruct((S, D), jnp.float32),
    )(delta_tok, base_out)


def kernel(x, Wg, Wu, Wd, Wr, Ag, Bg, Au, Bu, Ad, Bd):
    b, s, d = x.shape
    xf = x.reshape(-1, d)

    gate_p, up_p, x_p, base_out, logits = _base_mlp(
        xf, Wg.astype(jnp.bfloat16), Wu.astype(jnp.bfloat16),
        Wd.astype(jnp.bfloat16), Wr.astype(jnp.bfloat16))

    probs = jax.nn.softmax(logits, axis=-1)
    rw, sel = jax.lax.top_k(probs, TOPK)
    tile_expert, slot_w, ppos_flat = _dispatch(sel, rw)

    tok_of_pair = jnp.arange(NPAIR, dtype=jnp.int32) // TOPK
    gs_s, us_s, xs_s = _stage_gather(gate_p, up_p, x_p, tok_of_pair,
                                     ppos_flat)

    perm = jnp.asarray(_PERM_FF)
    AgAub = jnp.concatenate([Ag, Au], axis=1).astype(jnp.bfloat16)
    Adb = Ad[:, :, perm].astype(jnp.bfloat16)
    BgTb = jnp.swapaxes(Bg, 1, 2)[:, :, perm].astype(jnp.bfloat16)
    BuTb = jnp.swapaxes(Bu, 1, 2)[:, :, perm].astype(jnp.bfloat16)
    BdTb = jnp.swapaxes(Bd, 1, 2).astype(jnp.bfloat16)

    delta = _expert_deltas(tile_expert, gs_s, us_s, xs_s,
                           slot_w.reshape(P, 1), AgAub, BgTb, BuTb,
                           Adb, BdTb)

    delta_tok = _delta_to_token_order(delta, ppos_flat)
    return _combine(delta_tok, base_out).reshape(b, s, d)
